# baseline (speedup 1.0000x reference)
import dataclasses

import jax
import jax.numpy as jnp
from jax import lax
from jax.experimental import pallas as pl
from jax.experimental.pallas import tpu as pltpu
from jax.experimental.pallas import tpu_sc as plsc

E = 8
TOPK = 2
D = 1024
DF = 1024
M = 2048
BM = 256
NB = 24
TPAD = NB * BM
NEG = -1e30


def _route_body(logits_ref, pos_ref, wts_ref, meta_ref):
    lg = logits_ref[...]
    ids = lax.broadcasted_iota(jnp.int32, (M, E), 1)
    mx = jnp.max(lg, axis=1, keepdims=True)
    a1 = jnp.min(jnp.where(lg == mx, ids, E), axis=1, keepdims=True)
    oh0 = ids == a1
    lg2 = jnp.where(oh0, NEG, lg)
    mx2 = jnp.max(lg2, axis=1, keepdims=True)
    a2 = jnp.min(jnp.where(lg2 == mx2, ids, E), axis=1, keepdims=True)
    oh1 = ids == a2
    e2 = jnp.exp(mx2 - mx)
    s = 1.0 + e2
    wts_ref[:, 0:16] = jnp.broadcast_to(1.0 / s, (M, 16))
    wts_ref[:, 16:32] = jnp.broadcast_to(e2 / s, (M, 16))

    r = (oh0 | oh1).astype(jnp.float32)
    sft = 1
    while sft < M:
        r = r + jnp.concatenate(
            [jnp.zeros((sft, E), jnp.float32), r[: M - sft]], axis=0)
        sft *= 2
    ci = r[M - 1 : M, :].astype(jnp.int32)
    pc = ((ci + (BM - 1)) // BM) * BM
    inc = pc
    sft = 1
    while sft < E:
        inc = inc + jnp.concatenate(
            [jnp.zeros((1, sft), jnp.int32), inc[:, : E - sft]], axis=1)
        sft *= 2
    pstart = inc - pc
    base = pstart + r.astype(jnp.int32) - 1
    pos_ref[:, 0:1] = jnp.sum(jnp.where(oh0, base, 0), axis=1, keepdims=True)
    pos_ref[:, 1:2] = jnp.sum(jnp.where(oh1, base, 0), axis=1, keepdims=True)

    bidx = lax.broadcasted_iota(jnp.int32, (NB, E), 0) * BM
    ends_pad = pstart + pc
    bg = jnp.sum((ends_pad <= bidx).astype(jnp.int32), axis=1, keepdims=True)
    meta_ref[:, 0:1] = jnp.minimum(bg, E - 1)
    real_end = pstart + ci
    meta_ref[:, 1:2] = jnp.sum(
        ((pstart <= bidx) & (bidx < real_end)).astype(jnp.int32),
        axis=1, keepdims=True)


def _route(router_logits):
    return pl.pallas_call(
        _route_body,
        out_shape=[
            jax.ShapeDtypeStruct((M, TOPK), jnp.int32),
            jax.ShapeDtypeStruct((M, 16 * TOPK), jnp.float32),
            jax.ShapeDtypeStruct((NB, 2), jnp.int32),
        ],
    )(router_logits)


def _gmm_body(meta_ref, xs_ref, w1g_ref, w1u_ref, w2_ref, ys_ref, g_s, a_s):
    i = pl.program_id(0)
    p = pl.program_id(1)
    act = meta_ref[i, 1] == 1

    @pl.when(act & (p == 0))
    def _():
        g_s[...] = lax.dot_general(xs_ref[...], w1g_ref[0],
                                   (((1,), (1,)), ((), ())),
                                   preferred_element_type=jnp.float32)

    @pl.when(act & (p == 1))
    def _():
        g = g_s[...]
        u = lax.dot_general(xs_ref[...], w1u_ref[0],
                            (((1,), (1,)), ((), ())),
                            preferred_element_type=jnp.float32)
        a_s[...] = g * (1.0 / (1.0 + jnp.exp(-g))) * u

    @pl.when(act & (p == 2))
    def _():
        ys_ref[...] = lax.dot_general(a_s[...], w2_ref[0],
                                      (((1,), (1,)), ((), ())),
                                      preferred_element_type=jnp.float32)


def _gmm(meta, xs, w1, w2):
    grid_spec = pltpu.PrefetchScalarGridSpec(
        num_scalar_prefetch=1,
        grid=(NB, 3),
        in_specs=[
            pl.BlockSpec((BM, D), lambda i, p, meta: (i, 0)),
            pl.BlockSpec((1, DF, D), lambda i, p, meta: (meta[i, 0], 0, 0)),
            pl.BlockSpec((1, DF, D), lambda i, p, meta: (meta[i, 0], 1, 0)),
            pl.BlockSpec((1, D, DF), lambda i, p, meta: (meta[i, 0], 0, 0)),
        ],
        out_specs=pl.BlockSpec((BM, D), lambda i, p, meta: (i, 0)),
        scratch_shapes=[
            pltpu.VMEM((BM, DF), jnp.float32),
            pltpu.VMEM((BM, DF), jnp.float32),
        ],
    )
    return pl.pallas_call(
        _gmm_body,
        grid_spec=grid_spec,
        out_shape=jax.ShapeDtypeStruct((TPAD, D), jnp.float32),
        compiler_params=pltpu.CompilerParams(
            dimension_semantics=("arbitrary", "arbitrary")),
    )(meta, xs, w1, w1, w2)


def _sc_params():
    cp = pltpu.CompilerParams()
    if "needs_layout_passes" in pltpu.CompilerParams.__dataclass_fields__:
        cp = dataclasses.replace(cp, needs_layout_passes=False)
    return cp


def _sc_mesh():
    info = plsc.get_sparse_core_info()
    return (plsc.VectorSubcoreMesh(core_axis_name="c", subcore_axis_name="s"),
            info.num_cores, info.num_subcores)


def _dispatch(x, pos_flat):
    mesh, nc, ns = _sc_mesh()
    nw = nc * ns
    tpw = M // nw
    ppw = tpw * TOPK

    def body(x_hbm, posf_hbm, xs_hbm, pf_v, evo_v, rows_v, sgr, ssa, ssb):
        wid = lax.axis_index("s") * nc + lax.axis_index("c")
        tok_base = wid * tpw
        pair_base = tok_base * TOPK
        gr = pltpu.async_copy(x_hbm.at[pl.ds(tok_base, tpw)], rows_v, sgr)
        pltpu.sync_copy(posf_hbm.at[pl.ds(pair_base, ppw)], pf_v)
        lane = lax.broadcasted_iota(jnp.int32, (16,), 0)
        for h in range(2):
            for k in range(tpw // 16):
                g = plsc.load_gather(pf_v, [k * 32 + 2 * lane + h])
                evo_v[h, pl.ds(k * 16, 16)] = g
        gr.wait()
        s0 = pltpu.async_copy(rows_v, xs_hbm.at[evo_v.at[0]], ssa)
        s1 = pltpu.async_copy(rows_v, xs_hbm.at[evo_v.at[1]], ssb)
        s0.wait()
        s1.wait()

    f = pl.kernel(
        body,
        out_type=jax.ShapeDtypeStruct((TPAD, D), jnp.float32),
        mesh=mesh,
        scratch_types=[
            pltpu.VMEM((ppw,), jnp.int32),
            pltpu.VMEM((2, tpw), jnp.int32),
            pltpu.VMEM((tpw, D), jnp.float32),
            pltpu.SemaphoreType.DMA,
            pltpu.SemaphoreType.DMA,
            pltpu.SemaphoreType.DMA,
        ],
        compiler_params=_sc_params(),
    )
    return f(x, pos_flat)


def _combine(ys, pos_flat, wtsb):
    mesh, nc, ns = _sc_mesh()
    nw = nc * ns
    tpw = M // nw
    ppw = tpw * TOPK
    sub = 16
    nsub = tpw // sub

    def body(ys_hbm, posf_hbm, wtsb_hbm, out_hbm,
             idx_v, wv_v, buf_a, buf_b, out_a, out_b, sga, sgb, swa, swb):
        wid = lax.axis_index("s") * nc + lax.axis_index("c")
        tok_base = wid * tpw
        pair_base = tok_base * TOPK
        pltpu.sync_copy(wtsb_hbm.at[pl.ds(pair_base, ppw), :], wv_v)
        for ch in range(nsub):
            pltpu.sync_copy(
                posf_hbm.at[pl.ds(pair_base + ch * sub * TOPK, sub * TOPK)],
                idx_v.at[ch])
        bufs = (buf_a, buf_b)
        outs = (out_a, out_b)
        gsem = (sga, sgb)
        wsem = (swa, swb)
        gws = [None] * nsub
        wws = [None] * nsub
        gws[0] = pltpu.async_copy(ys_hbm.at[idx_v.at[0]], bufs[0], gsem[0])
        for ch in range(nsub):
            b = ch % 2
            gws[ch].wait()
            if ch + 1 < nsub:
                gws[ch + 1] = pltpu.async_copy(
                    ys_hbm.at[idx_v.at[ch + 1]], bufs[(ch + 1) % 2],
                    gsem[(ch + 1) % 2])
            if ch >= 2:
                wws[ch - 2].wait()

            @pl.loop(0, sub)
            def _(i):
                w0 = wv_v[ch * sub * TOPK + 2 * i, :]
                w1 = wv_v[ch * sub * TOPK + 2 * i + 1, :]
                for d in range(D // 16):
                    sl = pl.ds(d * 16, 16)
                    outs[b][i, sl] = (bufs[b][2 * i, sl] * w0
                                      + bufs[b][2 * i + 1, sl] * w1)

            wws[ch] = pltpu.async_copy(
                outs[b], out_hbm.at[pl.ds(tok_base + ch * sub, sub)], wsem[b])
        wws[nsub - 2].wait()
        wws[nsub - 1].wait()

    f = pl.kernel(
        body,
        out_type=jax.ShapeDtypeStruct((M, D), jnp.float32),
        mesh=mesh,
        scratch_types=[
            pltpu.VMEM((nsub, sub * TOPK), jnp.int32),
            pltpu.VMEM((ppw, 16), jnp.float32),
            pltpu.VMEM((sub * TOPK, D), jnp.float32),
            pltpu.VMEM((sub * TOPK, D), jnp.float32),
            pltpu.VMEM((sub, D), jnp.float32),
            pltpu.VMEM((sub, D), jnp.float32),
            pltpu.SemaphoreType.DMA,
            pltpu.SemaphoreType.DMA,
            pltpu.SemaphoreType.DMA,
            pltpu.SemaphoreType.DMA,
        ],
    )
    return f(ys, pos_flat, wtsb)


def kernel(x, router_logits, w1, w2):
    pos, wtsb, meta = _route(router_logits)
    pos_flat = pos.reshape(M * TOPK)
    wtsb_flat = wtsb.reshape(M * TOPK, 16)
    xs = _dispatch(x, pos_flat)
    ys = _gmm(meta, xs, w1, w2)
    return _combine(ys, pos_flat, wtsb_flat)


# speedup vs baseline: 1.1778x; 1.1778x over previous
import dataclasses

import jax
import jax.numpy as jnp
from jax import lax
from jax.experimental import pallas as pl
from jax.experimental.pallas import tpu as pltpu
from jax.experimental.pallas import tpu_sc as plsc

E = 8
TOPK = 2
D = 1024
DF = 1024
M = 2048
BM = 256
NB = 24
TPAD = NB * BM
NEG = -1e30


def _route_body(logits_ref, pos_ref, wts_ref, meta_ref):
    lg = logits_ref[...]
    ids = lax.broadcasted_iota(jnp.int32, (M, E), 1)
    mx = jnp.max(lg, axis=1, keepdims=True)
    a1 = jnp.min(jnp.where(lg == mx, ids, E), axis=1, keepdims=True)
    oh0 = ids == a1
    lg2 = jnp.where(oh0, NEG, lg)
    mx2 = jnp.max(lg2, axis=1, keepdims=True)
    a2 = jnp.min(jnp.where(lg2 == mx2, ids, E), axis=1, keepdims=True)
    oh1 = ids == a2
    e2 = jnp.exp(mx2 - mx)
    s = 1.0 + e2
    wts_ref[:, 0:16] = jnp.broadcast_to(1.0 / s, (M, 16))
    wts_ref[:, 16:32] = jnp.broadcast_to(e2 / s, (M, 16))

    r = (oh0 | oh1).astype(jnp.float32)
    sft = 1
    while sft < M:
        r = r + jnp.concatenate(
            [jnp.zeros((sft, E), jnp.float32), r[: M - sft]], axis=0)
        sft *= 2
    ci = r[M - 1 : M, :].astype(jnp.int32)
    pc = ((ci + (BM - 1)) // BM) * BM
    inc = pc
    sft = 1
    while sft < E:
        inc = inc + jnp.concatenate(
            [jnp.zeros((1, sft), jnp.int32), inc[:, : E - sft]], axis=1)
        sft *= 2
    pstart = inc - pc
    base = pstart + r.astype(jnp.int32) - 1
    pos_ref[:, 0:1] = jnp.sum(jnp.where(oh0, base, 0), axis=1, keepdims=True)
    pos_ref[:, 1:2] = jnp.sum(jnp.where(oh1, base, 0), axis=1, keepdims=True)

    bidx = lax.broadcasted_iota(jnp.int32, (NB, E), 0) * BM
    ends_pad = pstart + pc
    bg = jnp.sum((ends_pad <= bidx).astype(jnp.int32), axis=1, keepdims=True)
    meta_ref[:, 0:1] = jnp.minimum(bg, E - 1)
    real_end = pstart + ci
    meta_ref[:, 1:2] = jnp.sum(
        ((pstart <= bidx) & (bidx < real_end)).astype(jnp.int32),
        axis=1, keepdims=True)


def _route(router_logits):
    return pl.pallas_call(
        _route_body,
        out_shape=[
            jax.ShapeDtypeStruct((M, TOPK), jnp.int32),
            jax.ShapeDtypeStruct((M, 16 * TOPK), jnp.float32),
            jax.ShapeDtypeStruct((NB, 2), jnp.int32),
        ],
    )(router_logits)


def _gmm_body(meta_ref, xs_ref, w1_ref, w2_ref, ys_ref):
    i = pl.program_id(0)

    @pl.when(meta_ref[i, 1] == 1)
    def _():
        xb = xs_ref[...]
        wg = w1_ref[0, pl.ds(0, DF), :]
        wu = w1_ref[0, pl.ds(DF, DF), :]
        g = lax.dot_general(xb, wg, (((1,), (1,)), ((), ())),
                            preferred_element_type=jnp.float32)
        u = lax.dot_general(xb, wu, (((1,), (1,)), ((), ())),
                            preferred_element_type=jnp.float32)
        a = g * (1.0 / (1.0 + jnp.exp(-g))) * u
        ys_ref[...] = lax.dot_general(a, w2_ref[0], (((1,), (1,)), ((), ())),
                                      preferred_element_type=jnp.float32)


def _gmm(meta, xs, w1, w2):
    grid_spec = pltpu.PrefetchScalarGridSpec(
        num_scalar_prefetch=1,
        grid=(NB,),
        in_specs=[
            pl.BlockSpec((BM, D), lambda i, meta: (i, 0)),
            pl.BlockSpec((1, 2 * DF, D), lambda i, meta: (meta[i, 0], 0, 0)),
            pl.BlockSpec((1, D, DF), lambda i, meta: (meta[i, 0], 0, 0)),
        ],
        out_specs=pl.BlockSpec((BM, D), lambda i, meta: (i, 0)),
    )
    return pl.pallas_call(
        _gmm_body,
        grid_spec=grid_spec,
        out_shape=jax.ShapeDtypeStruct((TPAD, D), jnp.float32),
        compiler_params=pltpu.CompilerParams(
            dimension_semantics=("arbitrary",)),
    )(meta, xs, w1, w2)


def _sc_params():
    cp = pltpu.CompilerParams()
    if "needs_layout_passes" in pltpu.CompilerParams.__dataclass_fields__:
        cp = dataclasses.replace(cp, needs_layout_passes=False)
    return cp


def _sc_mesh():
    info = plsc.get_sparse_core_info()
    return (plsc.VectorSubcoreMesh(core_axis_name="c", subcore_axis_name="s"),
            info.num_cores, info.num_subcores)


def _dispatch(x, pos_flat):
    mesh, nc, ns = _sc_mesh()
    nw = nc * ns
    tpw = M // nw
    ppw = tpw * TOPK

    def body(x_hbm, posf_hbm, xs_hbm, pf_v, evo_v, rows_v, sgr, ssa, ssb):
        wid = lax.axis_index("s") * nc + lax.axis_index("c")
        tok_base = wid * tpw
        pair_base = tok_base * TOPK
        gr = pltpu.async_copy(x_hbm.at[pl.ds(tok_base, tpw)], rows_v, sgr)
        pltpu.sync_copy(posf_hbm.at[pl.ds(pair_base, ppw)], pf_v)
        lane = lax.broadcasted_iota(jnp.int32, (16,), 0)
        for h in range(2):
            for k in range(tpw // 16):
                g = plsc.load_gather(pf_v, [k * 32 + 2 * lane + h])
                evo_v[h, pl.ds(k * 16, 16)] = g
        gr.wait()
        s0 = pltpu.async_copy(rows_v, xs_hbm.at[evo_v.at[0]], ssa)
        s1 = pltpu.async_copy(rows_v, xs_hbm.at[evo_v.at[1]], ssb)
        s0.wait()
        s1.wait()

    f = pl.kernel(
        body,
        out_type=jax.ShapeDtypeStruct((TPAD, D), jnp.float32),
        mesh=mesh,
        scratch_types=[
            pltpu.VMEM((ppw,), jnp.int32),
            pltpu.VMEM((2, tpw), jnp.int32),
            pltpu.VMEM((tpw, D), jnp.float32),
            pltpu.SemaphoreType.DMA,
            pltpu.SemaphoreType.DMA,
            pltpu.SemaphoreType.DMA,
        ],
        compiler_params=_sc_params(),
    )
    return f(x, pos_flat)


def _combine(ys, pos_flat, wtsb):
    mesh, nc, ns = _sc_mesh()
    nw = nc * ns
    tpw = M // nw
    ppw = tpw * TOPK
    sub = 16
    nsub = tpw // sub

    def body(ys_hbm, posf_hbm, wtsb_hbm, out_hbm,
             idx_v, wv_v, buf_a, buf_b, out_a, out_b, sga, sgb, swa, swb):
        wid = lax.axis_index("s") * nc + lax.axis_index("c")
        tok_base = wid * tpw
        pair_base = tok_base * TOPK
        pltpu.sync_copy(wtsb_hbm.at[pl.ds(pair_base, ppw), :], wv_v)
        for ch in range(nsub):
            pltpu.sync_copy(
                posf_hbm.at[pl.ds(pair_base + ch * sub * TOPK, sub * TOPK)],
                idx_v.at[ch])
        bufs = (buf_a, buf_b)
        outs = (out_a, out_b)
        gsem = (sga, sgb)
        wsem = (swa, swb)
        gws = [None] * nsub
        wws = [None] * nsub
        gws[0] = pltpu.async_copy(ys_hbm.at[idx_v.at[0]], bufs[0], gsem[0])
        for ch in range(nsub):
            b = ch % 2
            gws[ch].wait()
            if ch + 1 < nsub:
                gws[ch + 1] = pltpu.async_copy(
                    ys_hbm.at[idx_v.at[ch + 1]], bufs[(ch + 1) % 2],
                    gsem[(ch + 1) % 2])
            if ch >= 2:
                wws[ch - 2].wait()

            @pl.loop(0, sub)
            def _(i):
                w0 = wv_v[ch * sub * TOPK + 2 * i, :]
                w1 = wv_v[ch * sub * TOPK + 2 * i + 1, :]
                for d in range(D // 16):
                    sl = pl.ds(d * 16, 16)
                    outs[b][i, sl] = (bufs[b][2 * i, sl] * w0
                                      + bufs[b][2 * i + 1, sl] * w1)

            wws[ch] = pltpu.async_copy(
                outs[b], out_hbm.at[pl.ds(tok_base + ch * sub, sub)], wsem[b])
        wws[nsub - 2].wait()
        wws[nsub - 1].wait()

    f = pl.kernel(
        body,
        out_type=jax.ShapeDtypeStruct((M, D), jnp.float32),
        mesh=mesh,
        scratch_types=[
            pltpu.VMEM((nsub, sub * TOPK), jnp.int32),
            pltpu.VMEM((ppw, 16), jnp.float32),
            pltpu.VMEM((sub * TOPK, D), jnp.float32),
            pltpu.VMEM((sub * TOPK, D), jnp.float32),
            pltpu.VMEM((sub, D), jnp.float32),
            pltpu.VMEM((sub, D), jnp.float32),
            pltpu.SemaphoreType.DMA,
            pltpu.SemaphoreType.DMA,
            pltpu.SemaphoreType.DMA,
            pltpu.SemaphoreType.DMA,
        ],
    )
    return f(ys, pos_flat, wtsb)


def kernel(x, router_logits, w1, w2):
    pos, wtsb, meta = _route(router_logits)
    pos_flat = pos.reshape(M * TOPK)
    wtsb_flat = wtsb.reshape(M * TOPK, 16)
    xs = _dispatch(x, pos_flat)
    ys = _gmm(meta, xs, w1, w2)
    return _combine(ys, pos_flat, wtsb_flat)
